# baseline (speedup 1.0000x reference)
import functools

import jax
import jax.numpy as jnp
from jax import lax
from jax.experimental import pallas as pl
from jax.experimental.pallas import tpu as pltpu
from jax.experimental.pallas import tpu_sc as plsc

N = 10000
E = 320000
D = 128

NC = 2
NS = 16
NW = NC * NS

CH = 80
NCHT = E // CH
NCH_W = NCHT // NW
GRP = 25
NGRP = NCH_W // GRP
EPW = NCH_W * CH
ATILES = 10
ASTRIPE = N // ATILES
AZ = 40
DEG_T = 5
DEG_STRIPE = N // DEG_T

_f32 = jnp.float32
_mesh = plsc.VectorSubcoreMesh(core_axis_name="c", subcore_axis_name="s")


def _deg_body(src_hbm, dst_hbm, ones_hbm, z1_hbm, dout_hbm, din_hbm,
              idx_s, idx_d, ones_v, stage_v, do_sh, di_sh):
    cid = lax.axis_index("c")
    sid = lax.axis_index("s")
    wid = cid * NS + sid

    pltpu.sync_copy(src_hbm.at[wid], idx_s)
    pltpu.sync_copy(dst_hbm.at[wid], idx_d)
    pltpu.sync_copy(ones_hbm, ones_v)

    @pl.when(sid < DEG_T)
    def _zero():
        base = sid * DEG_STRIPE
        pltpu.sync_copy(z1_hbm, stage_v)
        pltpu.sync_copy(stage_v, do_sh.at[pl.ds(base, DEG_STRIPE)])
        pltpu.sync_copy(stage_v, di_sh.at[pl.ds(base, DEG_STRIPE)])

    plsc.subcore_barrier()

    def body(j, carry):
        pltpu.sync_copy(ones_v, do_sh.at[idx_s.at[j]], add=True)
        pltpu.sync_copy(ones_v, di_sh.at[idx_d.at[j]], add=True)
        return carry

    lax.fori_loop(0, NCH_W, body, 0)
    plsc.subcore_barrier()

    @pl.when(sid < DEG_T)
    def _out():
        base = sid * DEG_STRIPE
        pltpu.sync_copy(do_sh.at[pl.ds(base, DEG_STRIPE)], stage_v)
        pltpu.sync_copy(stage_v, dout_hbm.at[pl.ds(cid * N + base, DEG_STRIPE)])
        pltpu.sync_copy(di_sh.at[pl.ds(base, DEG_STRIPE)], stage_v)
        pltpu.sync_copy(stage_v, din_hbm.at[pl.ds(cid * N + base, DEG_STRIPE)])


_deg_call = functools.partial(
    pl.kernel,
    out_type=(jax.ShapeDtypeStruct((NC * N,), _f32),
              jax.ShapeDtypeStruct((NC * N,), _f32)),
    mesh=_mesh,
    scratch_types=[
        pltpu.VMEM((NCH_W, CH), jnp.int32),
        pltpu.VMEM((NCH_W, CH), jnp.int32),
        pltpu.VMEM((CH,), _f32),
        pltpu.VMEM((DEG_STRIPE,), _f32),
        pltpu.VMEM_SHARED((N,), _f32),
        pltpu.VMEM_SHARED((N,), _f32),
    ],
)(_deg_body)


def _agg_body(src_hbm, dst_hbm, table_hbm, z2_hbm, out_hbm,
              idx_s, idx_d, rows, agg_sh,
              sg0, sg1, sg2, sg3, st0, st1, st2, st3):
    cid = lax.axis_index("c")
    sid = lax.axis_index("s")
    wid = cid * NS + sid

    pltpu.sync_copy(z2_hbm, rows.at[0, pl.ds(0, AZ)])

    @pl.when(sid < ATILES)
    def _zero():
        for k in range(ASTRIPE // AZ):
            pltpu.sync_copy(rows.at[0, pl.ds(0, AZ)],
                            agg_sh.at[pl.ds(sid * ASTRIPE + k * AZ, AZ)])

    plsc.subcore_barrier()

    SG = (sg0, sg1, sg2, sg3)
    ST = (st0, st1, st2, st3)

    def start_g(l, b):
        pltpu.async_copy(table_hbm.at[idx_s.at[l]], rows.at[b], SG[b])

    def wait_g(l, b):
        pltpu.make_async_copy(table_hbm.at[idx_s.at[l]], rows.at[b],
                              SG[b]).wait()

    def start_s(l, b):
        pltpu.async_copy(rows.at[b], agg_sh.at[idx_d.at[l]], ST[b], add=True)

    def wait_s(l, b):
        pltpu.make_async_copy(rows.at[b], agg_sh.at[idx_d.at[l]],
                              ST[b]).wait()

    for g in range(NGRP):
        pltpu.sync_copy(src_hbm.at[wid, g], idx_s)
        pltpu.sync_copy(dst_hbm.at[wid, g], idx_d)
        start_g(0, 0)
        start_g(1, 1)
        wait_g(0, 0)
        start_s(0, 0)
        start_g(2, 2)
        wait_g(1, 1)
        start_s(1, 1)
        start_g(3, 3)

        def body(i, carry):
            l0 = 2 + 4 * i
            for t in range(4):
                l = l0 + t
                b = (2 + t) % 4
                wait_g(l, b)
                start_s(l, b)
                wait_s(l - 2, (b + 2) % 4)
                start_g(l + 2, (b + 2) % 4)
            return carry

        lax.fori_loop(0, (GRP - 5) // 4, body, 0)
        wait_g(GRP - 3, 2)
        start_s(GRP - 3, 2)
        wait_s(GRP - 5, 0)
        start_g(GRP - 1, 0)
        wait_g(GRP - 2, 3)
        start_s(GRP - 2, 3)
        wait_s(GRP - 4, 1)
        wait_g(GRP - 1, 0)
        start_s(GRP - 1, 0)
        wait_s(GRP - 3, 2)
        wait_s(GRP - 2, 3)
        wait_s(GRP - 1, 0)
    plsc.subcore_barrier()

    @pl.when(sid < ATILES)
    def _out():
        pltpu.sync_copy(agg_sh.at[pl.ds(sid * ASTRIPE, ASTRIPE)],
                        out_hbm.at[cid, pl.ds(sid * ASTRIPE, ASTRIPE)])


_agg_call = functools.partial(
    pl.kernel,
    out_type=jax.ShapeDtypeStruct((NC, N, D), _f32),
    mesh=_mesh,
    scratch_types=[
        pltpu.VMEM((GRP, CH), jnp.int32),
        pltpu.VMEM((GRP, CH), jnp.int32),
        pltpu.VMEM((4, CH, D), _f32),
        pltpu.VMEM_SHARED((N, D), _f32),
        pltpu.SemaphoreType.DMA,
        pltpu.SemaphoreType.DMA,
        pltpu.SemaphoreType.DMA,
        pltpu.SemaphoreType.DMA,
        pltpu.SemaphoreType.DMA,
        pltpu.SemaphoreType.DMA,
        pltpu.SemaphoreType.DMA,
        pltpu.SemaphoreType.DMA,
    ],
)(_agg_body)


BN = 1000


def _prep_body(x_ref, dop_ref, o_ref):
    do = jnp.maximum(dop_ref[0] + dop_ref[1], 1.0)
    o_ref[...] = x_ref[...] * lax.rsqrt(do)


_prep_call = pl.pallas_call(
    _prep_body,
    grid=(N // BN,),
    in_specs=[
        pl.BlockSpec((BN, D), lambda i: (i, 0)),
        pl.BlockSpec((NC, BN, 1), lambda i: (0, i, 0)),
    ],
    out_specs=pl.BlockSpec((BN, D), lambda i: (i, 0)),
    out_shape=jax.ShapeDtypeStruct((N, D), _f32),
)


def _dense_math(aggp, dip, x, w, b, g, be):
    rs_in = lax.rsqrt(jnp.maximum(dip[0] + dip[1], 1.0))
    a = (aggp[0] + aggp[1]) * rs_in
    h = jnp.dot(a, w[...], preferred_element_type=_f32) + b[...]
    mu = jnp.mean(h, axis=1, keepdims=True)
    var = jnp.mean((h - mu) ** 2, axis=1, keepdims=True)
    h = (h - mu) * lax.rsqrt(var + 1e-5) * g[...] + be[...]
    return jnp.maximum(h, 0.0) + x[...]


def _dense_body2(aggp, dip, dop, x, w, b, g, be, out_h, out_s):
    h = _dense_math(aggp, dip, x, w, b, g, be)
    out_h[...] = h
    out_s[...] = h * lax.rsqrt(jnp.maximum(dop[0] + dop[1], 1.0))


def _dense_body1(aggp, dip, dop, x, w, b, g, be, out_h):
    out_h[...] = _dense_math(aggp, dip, x, w, b, g, be)


def _make_dense(n_out):
    body = _dense_body2 if n_out == 2 else _dense_body1
    out_shape = [jax.ShapeDtypeStruct((N, D), _f32)] * n_out
    out_specs = [pl.BlockSpec((BN, D), lambda i: (i, 0))] * n_out
    if n_out == 1:
        out_shape, out_specs = out_shape[0], out_specs[0]
    return pl.pallas_call(
        body,
        grid=(N // BN,),
        in_specs=[
            pl.BlockSpec((NC, BN, D), lambda i: (0, i, 0)),
            pl.BlockSpec((NC, BN, 1), lambda i: (0, i, 0)),
            pl.BlockSpec((NC, BN, 1), lambda i: (0, i, 0)),
            pl.BlockSpec((BN, D), lambda i: (i, 0)),
            pl.BlockSpec((D, D), lambda i: (0, 0)),
            pl.BlockSpec((1, D), lambda i: (0, 0)),
            pl.BlockSpec((1, D), lambda i: (0, 0)),
            pl.BlockSpec((1, D), lambda i: (0, 0)),
        ],
        out_specs=out_specs,
        out_shape=out_shape,
    )


_dense2 = _make_dense(2)
_dense1 = _make_dense(1)


def kernel(features, edge_index, W1, b1, ln_g1, ln_b1, W2, b2, ln_g2, ln_b2):
    src4d = edge_index[0].reshape(NW, NGRP, GRP, CH)
    dst4d = edge_index[1].reshape(NW, NGRP, GRP, CH)
    ones1 = jnp.ones((CH,), _f32)
    z1 = jnp.zeros((DEG_STRIPE,), _f32)
    z2 = jnp.zeros((AZ, D), _f32)

    src3d = edge_index[0].reshape(NW, NCH_W, CH)
    dst3d = edge_index[1].reshape(NW, NCH_W, CH)
    dout_p, din_p = _deg_call(src3d, dst3d, ones1, z1)
    dop = dout_p.reshape(NC, N, 1)
    dip = din_p.reshape(NC, N, 1)

    scaled = _prep_call(features, dop)
    aggp = _agg_call(src4d, dst4d, scaled, z2)
    h1, scaled2 = _dense2(aggp, dip, dop, features,
                          W1, b1.reshape(1, D), ln_g1.reshape(1, D),
                          ln_b1.reshape(1, D))
    aggp2 = _agg_call(src4d, dst4d, scaled2, z2)
    return _dense1(aggp2, dip, dop, h1,
                   W2, b2.reshape(1, D), ln_g2.reshape(1, D),
                   ln_b2.reshape(1, D))


# speedup vs baseline: 1.1234x; 1.1234x over previous
import functools

import jax
import jax.numpy as jnp
from jax import lax
from jax.experimental import pallas as pl
from jax.experimental.pallas import tpu as pltpu
from jax.experimental.pallas import tpu_sc as plsc

N = 10000
E = 320000
D = 128

NC = 2
NS = 16
NW = NC * NS

CH = 125
NCHT = E // CH
NCH_W = NCHT // NW
GRP = 40
NGRP = NCH_W // GRP
EPW = NCH_W * CH
ATILES = 10
ASTRIPE = N // ATILES
AZ = 40
DEG_T = 5
DEG_STRIPE = N // DEG_T

_f32 = jnp.float32
_mesh = plsc.VectorSubcoreMesh(core_axis_name="c", subcore_axis_name="s")


def _deg_body(src_hbm, dst_hbm, ones_hbm, z1_hbm, dout_hbm, din_hbm,
              idx_s, idx_d, ones_v, stage_v, do_sh, di_sh, sda, sdb):
    cid = lax.axis_index("c")
    sid = lax.axis_index("s")
    wid = cid * NS + sid

    pltpu.sync_copy(src_hbm.at[wid], idx_s)
    pltpu.sync_copy(dst_hbm.at[wid], idx_d)
    pltpu.sync_copy(ones_hbm, ones_v)

    @pl.when(sid < DEG_T)
    def _zero():
        base = sid * DEG_STRIPE
        pltpu.sync_copy(z1_hbm, stage_v)
        pltpu.sync_copy(stage_v, do_sh.at[pl.ds(base, DEG_STRIPE)])
        pltpu.sync_copy(stage_v, di_sh.at[pl.ds(base, DEG_STRIPE)])

    plsc.subcore_barrier()

    def body(j, carry):
        d1 = pltpu.async_copy(ones_v, do_sh.at[idx_s.at[j]], sda, add=True)
        d2 = pltpu.async_copy(ones_v, di_sh.at[idx_d.at[j]], sdb, add=True)
        d1.wait()
        d2.wait()
        return carry

    lax.fori_loop(0, NCH_W, body, 0)
    plsc.subcore_barrier()

    @pl.when(sid < DEG_T)
    def _out():
        base = sid * DEG_STRIPE
        pltpu.sync_copy(do_sh.at[pl.ds(base, DEG_STRIPE)], stage_v)
        pltpu.sync_copy(stage_v, dout_hbm.at[pl.ds(cid * N + base, DEG_STRIPE)])
        pltpu.sync_copy(di_sh.at[pl.ds(base, DEG_STRIPE)], stage_v)
        pltpu.sync_copy(stage_v, din_hbm.at[pl.ds(cid * N + base, DEG_STRIPE)])


_deg_call = functools.partial(
    pl.kernel,
    out_type=(jax.ShapeDtypeStruct((NC * N,), _f32),
              jax.ShapeDtypeStruct((NC * N,), _f32)),
    mesh=_mesh,
    scratch_types=[
        pltpu.VMEM((NCH_W, CH), jnp.int32),
        pltpu.VMEM((NCH_W, CH), jnp.int32),
        pltpu.VMEM((CH,), _f32),
        pltpu.VMEM((DEG_STRIPE,), _f32),
        pltpu.VMEM_SHARED((N,), _f32),
        pltpu.VMEM_SHARED((N,), _f32),
        pltpu.SemaphoreType.DMA,
        pltpu.SemaphoreType.DMA,
    ],
)(_deg_body)


def _agg_body(src_hbm, dst_hbm, table_hbm, z2_hbm, out_hbm,
              idx_s, idx_d, rows, agg_sh, sem0, sem1):
    cid = lax.axis_index("c")
    sid = lax.axis_index("s")
    wid = cid * NS + sid

    pltpu.sync_copy(dst_hbm.at[wid], idx_d)
    pltpu.sync_copy(z2_hbm, rows.at[0, pl.ds(0, AZ)])

    @pl.when(sid < ATILES)
    def _zero():
        for k in range(ASTRIPE // AZ):
            pltpu.sync_copy(rows.at[0, pl.ds(0, AZ)],
                            agg_sh.at[pl.ds(sid * ASTRIPE + k * AZ, AZ)])

    plsc.subcore_barrier()

    sems = (sem0, sem1)

    def start(l, b):
        pltpu.async_copy(table_hbm.at[idx_s.at[l]], rows.at[b], sems[b])

    def finish(l, j, b):
        pltpu.make_async_copy(table_hbm.at[idx_s.at[l]], rows.at[b],
                              sems[b]).wait()
        pltpu.sync_copy(rows.at[b], agg_sh.at[idx_d.at[j]], add=True)

    for g in range(NGRP):
        pltpu.sync_copy(src_hbm.at[wid, g], idx_s)
        start(0, 0)
        start(1, 1)

        def body(i, carry, g=g):
            l0 = 2 * i
            finish(l0, g * GRP + l0, 0)
            start(l0 + 2, 0)
            finish(l0 + 1, g * GRP + l0 + 1, 1)
            start(l0 + 3, 1)
            return carry

        lax.fori_loop(0, (GRP - 4) // 2 + 1, body, 0)
        finish(GRP - 2, g * GRP + GRP - 2, 0)
        finish(GRP - 1, g * GRP + GRP - 1, 1)
    plsc.subcore_barrier()

    @pl.when(sid < ATILES)
    def _out():
        pltpu.sync_copy(agg_sh.at[pl.ds(sid * ASTRIPE, ASTRIPE)],
                        out_hbm.at[cid, pl.ds(sid * ASTRIPE, ASTRIPE)])


_agg_call = functools.partial(
    pl.kernel,
    out_type=jax.ShapeDtypeStruct((NC, N, D), _f32),
    mesh=_mesh,
    scratch_types=[
        pltpu.VMEM((GRP, CH), jnp.int32),
        pltpu.VMEM((NCH_W, CH), jnp.int32),
        pltpu.VMEM((2, CH, D), _f32),
        pltpu.VMEM_SHARED((N, D), _f32),
        pltpu.SemaphoreType.DMA,
        pltpu.SemaphoreType.DMA,
    ],
)(_agg_body)


BN = 1000


def _prep_body(x_ref, dop_ref, o_ref):
    do = jnp.maximum(dop_ref[0] + dop_ref[1], 1.0)
    o_ref[...] = x_ref[...] * lax.rsqrt(do)


_prep_call = pl.pallas_call(
    _prep_body,
    grid=(N // BN,),
    in_specs=[
        pl.BlockSpec((BN, D), lambda i: (i, 0)),
        pl.BlockSpec((NC, BN, 1), lambda i: (0, i, 0)),
    ],
    out_specs=pl.BlockSpec((BN, D), lambda i: (i, 0)),
    out_shape=jax.ShapeDtypeStruct((N, D), _f32),
)


def _dense_math(aggp, dip, x, w, b, g, be):
    rs_in = lax.rsqrt(jnp.maximum(dip[0] + dip[1], 1.0))
    a = (aggp[0] + aggp[1]) * rs_in
    h = jnp.dot(a, w[...], preferred_element_type=_f32) + b[...]
    mu = jnp.mean(h, axis=1, keepdims=True)
    var = jnp.mean((h - mu) ** 2, axis=1, keepdims=True)
    h = (h - mu) * lax.rsqrt(var + 1e-5) * g[...] + be[...]
    return jnp.maximum(h, 0.0) + x[...]


def _dense_body2(aggp, dip, dop, x, w, b, g, be, out_h, out_s):
    h = _dense_math(aggp, dip, x, w, b, g, be)
    out_h[...] = h
    out_s[...] = h * lax.rsqrt(jnp.maximum(dop[0] + dop[1], 1.0))


def _dense_body1(aggp, dip, dop, x, w, b, g, be, out_h):
    out_h[...] = _dense_math(aggp, dip, x, w, b, g, be)


def _make_dense(n_out):
    body = _dense_body2 if n_out == 2 else _dense_body1
    out_shape = [jax.ShapeDtypeStruct((N, D), _f32)] * n_out
    out_specs = [pl.BlockSpec((BN, D), lambda i: (i, 0))] * n_out
    if n_out == 1:
        out_shape, out_specs = out_shape[0], out_specs[0]
    return pl.pallas_call(
        body,
        grid=(N // BN,),
        in_specs=[
            pl.BlockSpec((NC, BN, D), lambda i: (0, i, 0)),
            pl.BlockSpec((NC, BN, 1), lambda i: (0, i, 0)),
            pl.BlockSpec((NC, BN, 1), lambda i: (0, i, 0)),
            pl.BlockSpec((BN, D), lambda i: (i, 0)),
            pl.BlockSpec((D, D), lambda i: (0, 0)),
            pl.BlockSpec((1, D), lambda i: (0, 0)),
            pl.BlockSpec((1, D), lambda i: (0, 0)),
            pl.BlockSpec((1, D), lambda i: (0, 0)),
        ],
        out_specs=out_specs,
        out_shape=out_shape,
    )


_dense2 = _make_dense(2)
_dense1 = _make_dense(1)


def kernel(features, edge_index, W1, b1, ln_g1, ln_b1, W2, b2, ln_g2, ln_b2):
    src4d = edge_index[0].reshape(NW, NGRP, GRP, CH)
    ones1 = jnp.ones((CH,), _f32)
    z1 = jnp.zeros((DEG_STRIPE,), _f32)
    z2 = jnp.zeros((AZ, D), _f32)

    src3d = edge_index[0].reshape(NW, NCH_W, CH)
    dst3d = edge_index[1].reshape(NW, NCH_W, CH)
    dout_p, din_p = _deg_call(src3d, dst3d, ones1, z1)
    dop = dout_p.reshape(NC, N, 1)
    dip = din_p.reshape(NC, N, 1)

    scaled = _prep_call(features, dop)
    aggp = _agg_call(src4d, dst3d, scaled, z2)
    h1, scaled2 = _dense2(aggp, dip, dop, features,
                          W1, b1.reshape(1, D), ln_g1.reshape(1, D),
                          ln_b1.reshape(1, D))
    aggp2 = _agg_call(src4d, dst3d, scaled2, z2)
    return _dense1(aggp2, dip, dop, h1,
                   W2, b2.reshape(1, D), ln_g2.reshape(1, D),
                   ln_b2.reshape(1, D))


# speedup vs baseline: 1.1407x; 1.0153x over previous
import functools

import jax
import jax.numpy as jnp
from jax import lax
from jax.experimental import pallas as pl
from jax.experimental.pallas import tpu as pltpu
from jax.experimental.pallas import tpu_sc as plsc

N = 10000
E = 320000
D = 128

NC = 2
NS = 16
NW = NC * NS

CH = 125
NCHT = E // CH
NCH_W = NCHT // NW
GRP = 40
NGRP = NCH_W // GRP
EPW = NCH_W * CH
ATILES = 10
ASTRIPE = N // ATILES
AZ = 40
DEG_T = 5
DEG_STRIPE = N // DEG_T

_f32 = jnp.float32
_mesh = plsc.VectorSubcoreMesh(core_axis_name="c", subcore_axis_name="s")


def _deg_body(src_hbm, dst_hbm, ones_hbm, z1_hbm, dout_hbm, din_hbm,
              idx_s, idx_d, ones_v, stage_v, do_sh, di_sh,
              sda, sdb, sdc, sdd):
    cid = lax.axis_index("c")
    sid = lax.axis_index("s")
    wid = cid * NS + sid

    pltpu.sync_copy(src_hbm.at[wid], idx_s)
    pltpu.sync_copy(dst_hbm.at[wid], idx_d)
    pltpu.sync_copy(ones_hbm, ones_v)

    @pl.when(sid < DEG_T)
    def _zero():
        base = sid * DEG_STRIPE
        pltpu.sync_copy(z1_hbm, stage_v)
        pltpu.sync_copy(stage_v, do_sh.at[pl.ds(base, DEG_STRIPE)])
        pltpu.sync_copy(stage_v, di_sh.at[pl.ds(base, DEG_STRIPE)])

    plsc.subcore_barrier()

    SDA = (sda, sdb)
    SDB = (sdc, sdd)

    def dstart(j, b):
        pltpu.async_copy(ones_v, do_sh.at[idx_s.at[j]], SDA[b], add=True)
        pltpu.async_copy(ones_v, di_sh.at[idx_d.at[j]], SDB[b], add=True)

    def dwait(j, b):
        pltpu.make_async_copy(ones_v, do_sh.at[idx_s.at[j]], SDA[b]).wait()
        pltpu.make_async_copy(ones_v, di_sh.at[idx_d.at[j]], SDB[b]).wait()

    dstart(0, 0)
    dstart(1, 1)

    def body(i, carry):
        j0 = 2 * i
        dwait(j0, 0)
        dstart(j0 + 2, 0)
        dwait(j0 + 1, 1)
        dstart(j0 + 3, 1)
        return carry

    lax.fori_loop(0, (NCH_W - 4) // 2 + 1, body, 0)
    dwait(NCH_W - 2, 0)
    dwait(NCH_W - 1, 1)
    plsc.subcore_barrier()

    @pl.when(sid < DEG_T)
    def _out():
        base = sid * DEG_STRIPE
        pltpu.sync_copy(do_sh.at[pl.ds(base, DEG_STRIPE)], stage_v)
        pltpu.sync_copy(stage_v, dout_hbm.at[pl.ds(cid * N + base, DEG_STRIPE)])
        pltpu.sync_copy(di_sh.at[pl.ds(base, DEG_STRIPE)], stage_v)
        pltpu.sync_copy(stage_v, din_hbm.at[pl.ds(cid * N + base, DEG_STRIPE)])


_deg_call = functools.partial(
    pl.kernel,
    out_type=(jax.ShapeDtypeStruct((NC * N,), _f32),
              jax.ShapeDtypeStruct((NC * N,), _f32)),
    mesh=_mesh,
    scratch_types=[
        pltpu.VMEM((NCH_W, CH), jnp.int32),
        pltpu.VMEM((NCH_W, CH), jnp.int32),
        pltpu.VMEM((CH,), _f32),
        pltpu.VMEM((DEG_STRIPE,), _f32),
        pltpu.VMEM_SHARED((N,), _f32),
        pltpu.VMEM_SHARED((N,), _f32),
        pltpu.SemaphoreType.DMA,
        pltpu.SemaphoreType.DMA,
        pltpu.SemaphoreType.DMA,
        pltpu.SemaphoreType.DMA,
    ],
)(_deg_body)


def _agg_body(src_hbm, dst_hbm, table_hbm, z2_hbm, out_hbm,
              idx_s, idx_d, rows, agg_sh, sem0, sem1):
    cid = lax.axis_index("c")
    sid = lax.axis_index("s")
    wid = cid * NS + sid

    pltpu.sync_copy(dst_hbm.at[wid], idx_d)
    pltpu.sync_copy(z2_hbm, rows.at[0, pl.ds(0, AZ)])

    @pl.when(sid < ATILES)
    def _zero():
        for k in range(ASTRIPE // AZ):
            pltpu.sync_copy(rows.at[0, pl.ds(0, AZ)],
                            agg_sh.at[pl.ds(sid * ASTRIPE + k * AZ, AZ)])

    plsc.subcore_barrier()

    sems = (sem0, sem1)

    def start(l, b):
        pltpu.async_copy(table_hbm.at[idx_s.at[l]], rows.at[b], sems[b])

    def finish(l, j, b):
        pltpu.make_async_copy(table_hbm.at[idx_s.at[l]], rows.at[b],
                              sems[b]).wait()
        pltpu.sync_copy(rows.at[b], agg_sh.at[idx_d.at[j]], add=True)

    for g in range(NGRP):
        pltpu.sync_copy(src_hbm.at[wid, g], idx_s)
        start(0, 0)
        start(1, 1)

        def body(i, carry, g=g):
            l0 = 2 * i
            finish(l0, g * GRP + l0, 0)
            start(l0 + 2, 0)
            finish(l0 + 1, g * GRP + l0 + 1, 1)
            start(l0 + 3, 1)
            return carry

        lax.fori_loop(0, (GRP - 4) // 2 + 1, body, 0)
        finish(GRP - 2, g * GRP + GRP - 2, 0)
        finish(GRP - 1, g * GRP + GRP - 1, 1)
    plsc.subcore_barrier()

    @pl.when(sid < ATILES)
    def _out():
        pltpu.sync_copy(agg_sh.at[pl.ds(sid * ASTRIPE, ASTRIPE)],
                        out_hbm.at[cid, pl.ds(sid * ASTRIPE, ASTRIPE)])


_agg_call = functools.partial(
    pl.kernel,
    out_type=jax.ShapeDtypeStruct((NC, N, D), _f32),
    mesh=_mesh,
    scratch_types=[
        pltpu.VMEM((GRP, CH), jnp.int32),
        pltpu.VMEM((NCH_W, CH), jnp.int32),
        pltpu.VMEM((2, CH, D), _f32),
        pltpu.VMEM_SHARED((N, D), _f32),
        pltpu.SemaphoreType.DMA,
        pltpu.SemaphoreType.DMA,
    ],
)(_agg_body)


BN = 1000


def _prep_body(x_ref, dop_ref, o_ref):
    do = jnp.maximum(dop_ref[0] + dop_ref[1], 1.0)
    o_ref[...] = x_ref[...] * lax.rsqrt(do)


_prep_call = pl.pallas_call(
    _prep_body,
    grid=(N // BN,),
    in_specs=[
        pl.BlockSpec((BN, D), lambda i: (i, 0)),
        pl.BlockSpec((NC, BN, 1), lambda i: (0, i, 0)),
    ],
    out_specs=pl.BlockSpec((BN, D), lambda i: (i, 0)),
    out_shape=jax.ShapeDtypeStruct((N, D), _f32),
)


def _dense_math(aggp, dip, x, w, b, g, be):
    rs_in = lax.rsqrt(jnp.maximum(dip[0] + dip[1], 1.0))
    a = (aggp[0] + aggp[1]) * rs_in
    h = jnp.dot(a, w[...], preferred_element_type=_f32) + b[...]
    mu = jnp.mean(h, axis=1, keepdims=True)
    var = jnp.mean((h - mu) ** 2, axis=1, keepdims=True)
    h = (h - mu) * lax.rsqrt(var + 1e-5) * g[...] + be[...]
    return jnp.maximum(h, 0.0) + x[...]


def _dense_body2(aggp, dip, dop, x, w, b, g, be, out_h, out_s):
    h = _dense_math(aggp, dip, x, w, b, g, be)
    out_h[...] = h
    out_s[...] = h * lax.rsqrt(jnp.maximum(dop[0] + dop[1], 1.0))


def _dense_body1(aggp, dip, dop, x, w, b, g, be, out_h):
    out_h[...] = _dense_math(aggp, dip, x, w, b, g, be)


def _make_dense(n_out):
    body = _dense_body2 if n_out == 2 else _dense_body1
    out_shape = [jax.ShapeDtypeStruct((N, D), _f32)] * n_out
    out_specs = [pl.BlockSpec((BN, D), lambda i: (i, 0))] * n_out
    if n_out == 1:
        out_shape, out_specs = out_shape[0], out_specs[0]
    return pl.pallas_call(
        body,
        grid=(N // BN,),
        in_specs=[
            pl.BlockSpec((NC, BN, D), lambda i: (0, i, 0)),
            pl.BlockSpec((NC, BN, 1), lambda i: (0, i, 0)),
            pl.BlockSpec((NC, BN, 1), lambda i: (0, i, 0)),
            pl.BlockSpec((BN, D), lambda i: (i, 0)),
            pl.BlockSpec((D, D), lambda i: (0, 0)),
            pl.BlockSpec((1, D), lambda i: (0, 0)),
            pl.BlockSpec((1, D), lambda i: (0, 0)),
            pl.BlockSpec((1, D), lambda i: (0, 0)),
        ],
        out_specs=out_specs,
        out_shape=out_shape,
    )


_dense2 = _make_dense(2)
_dense1 = _make_dense(1)


def kernel(features, edge_index, W1, b1, ln_g1, ln_b1, W2, b2, ln_g2, ln_b2):
    src4d = edge_index[0].reshape(NW, NGRP, GRP, CH)
    ones1 = jnp.ones((CH,), _f32)
    z1 = jnp.zeros((DEG_STRIPE,), _f32)
    z2 = jnp.zeros((AZ, D), _f32)

    src3d = edge_index[0].reshape(NW, NCH_W, CH)
    dst3d = edge_index[1].reshape(NW, NCH_W, CH)
    dout_p, din_p = _deg_call(src3d, dst3d, ones1, z1)
    dop = dout_p.reshape(NC, N, 1)
    dip = din_p.reshape(NC, N, 1)

    scaled = _prep_call(features, dop)
    aggp = _agg_call(src4d, dst3d, scaled, z2)
    h1, scaled2 = _dense2(aggp, dip, dop, features,
                          W1, b1.reshape(1, D), ln_g1.reshape(1, D),
                          ln_b1.reshape(1, D))
    aggp2 = _agg_call(src4d, dst3d, scaled2, z2)
    return _dense1(aggp2, dip, dop, h1,
                   W2, b2.reshape(1, D), ln_g2.reshape(1, D),
                   ln_b2.reshape(1, D))


# speedup vs baseline: 1.1464x; 1.0050x over previous
import functools

import jax
import jax.numpy as jnp
from jax import lax
from jax.experimental import pallas as pl
from jax.experimental.pallas import tpu as pltpu
from jax.experimental.pallas import tpu_sc as plsc

N = 10000
E = 320000
D = 128

NC = 2
NS = 16
NW = NC * NS

CH = 125
NCHT = E // CH
NCH_W = NCHT // NW
GRP = 40
NGRP = NCH_W // GRP
EPW = NCH_W * CH
ATILES = 10
ASTRIPE = N // ATILES
AZ = 40
DEG_T = 5
DEG_STRIPE = N // DEG_T

_f32 = jnp.float32
_mesh = plsc.VectorSubcoreMesh(core_axis_name="c", subcore_axis_name="s")


def _deg_body(src_hbm, dst_hbm, ones_hbm, z1_hbm, dout_hbm, din_hbm,
              idx_s, idx_d, ones_v, stage_v, do_sh, di_sh,
              sda, sdb, sdc, sdd):
    cid = lax.axis_index("c")
    sid = lax.axis_index("s")
    wid = cid * NS + sid

    pltpu.sync_copy(src_hbm.at[wid], idx_s)
    pltpu.sync_copy(dst_hbm.at[wid], idx_d)
    pltpu.sync_copy(ones_hbm, ones_v)

    @pl.when(sid < DEG_T)
    def _zero():
        base = sid * DEG_STRIPE
        pltpu.sync_copy(z1_hbm, stage_v)
        pltpu.sync_copy(stage_v, do_sh.at[pl.ds(base, DEG_STRIPE)])
        pltpu.sync_copy(stage_v, di_sh.at[pl.ds(base, DEG_STRIPE)])

    plsc.subcore_barrier()

    SDA = (sda, sdb)
    SDB = (sdc, sdd)

    def dstart(j, b):
        pltpu.async_copy(ones_v, do_sh.at[idx_s.at[j]], SDA[b], add=True)
        pltpu.async_copy(ones_v, di_sh.at[idx_d.at[j]], SDB[b], add=True)

    def dwait(j, b):
        pltpu.make_async_copy(ones_v, do_sh.at[idx_s.at[j]], SDA[b]).wait()
        pltpu.make_async_copy(ones_v, di_sh.at[idx_d.at[j]], SDB[b]).wait()

    dstart(0, 0)
    dstart(1, 1)

    def body(i, carry):
        j0 = 2 * i
        dwait(j0, 0)
        dstart(j0 + 2, 0)
        dwait(j0 + 1, 1)
        dstart(j0 + 3, 1)
        return carry

    lax.fori_loop(0, (NCH_W - 4) // 2 + 1, body, 0)
    dwait(NCH_W - 2, 0)
    dwait(NCH_W - 1, 1)
    plsc.subcore_barrier()

    @pl.when(sid < DEG_T)
    def _out():
        base = sid * DEG_STRIPE
        pltpu.sync_copy(do_sh.at[pl.ds(base, DEG_STRIPE)], stage_v)
        pltpu.sync_copy(stage_v, dout_hbm.at[pl.ds(cid * N + base, DEG_STRIPE)])
        pltpu.sync_copy(di_sh.at[pl.ds(base, DEG_STRIPE)], stage_v)
        pltpu.sync_copy(stage_v, din_hbm.at[pl.ds(cid * N + base, DEG_STRIPE)])


_deg_call = functools.partial(
    pl.kernel,
    out_type=(jax.ShapeDtypeStruct((NC * N,), _f32),
              jax.ShapeDtypeStruct((NC * N,), _f32)),
    mesh=_mesh,
    scratch_types=[
        pltpu.VMEM((NCH_W, CH), jnp.int32),
        pltpu.VMEM((NCH_W, CH), jnp.int32),
        pltpu.VMEM((CH,), _f32),
        pltpu.VMEM((DEG_STRIPE,), _f32),
        pltpu.VMEM_SHARED((N,), _f32),
        pltpu.VMEM_SHARED((N,), _f32),
        pltpu.SemaphoreType.DMA,
        pltpu.SemaphoreType.DMA,
        pltpu.SemaphoreType.DMA,
        pltpu.SemaphoreType.DMA,
    ],
)(_deg_body)


def _agg_body(src_hbm, dst_hbm, table_hbm, z2_hbm, out_hbm,
              idx_s, idx_d, rows, zbuf, agg_sh, sem0, sem1):
    cid = lax.axis_index("c")
    sid = lax.axis_index("s")
    wid = cid * NS + sid

    sems = (sem0, sem1)

    def start(l, b):
        pltpu.async_copy(table_hbm.at[idx_s.at[l]], rows.at[b], sems[b])

    def finish(l, b):
        pltpu.make_async_copy(table_hbm.at[idx_s.at[l]], rows.at[b],
                              sems[b]).wait()
        pltpu.sync_copy(rows.at[b], agg_sh.at[idx_d.at[l]], add=True)

    pltpu.sync_copy(src_hbm.at[wid, 0], idx_s)
    pltpu.sync_copy(dst_hbm.at[wid, 0], idx_d)
    start(0, 0)
    start(1, 1)
    pltpu.sync_copy(z2_hbm, zbuf)

    @pl.when(sid < ATILES)
    def _zero():
        for k in range(ASTRIPE // AZ):
            pltpu.sync_copy(zbuf, agg_sh.at[pl.ds(sid * ASTRIPE + k * AZ, AZ)])

    plsc.subcore_barrier()

    for g in range(NGRP):
        if g > 0:
            pltpu.sync_copy(src_hbm.at[wid, g], idx_s)
            pltpu.sync_copy(dst_hbm.at[wid, g], idx_d)
            start(0, 0)
            start(1, 1)

        def body(i, carry):
            l0 = 2 * i
            finish(l0, 0)
            start(l0 + 2, 0)
            finish(l0 + 1, 1)
            start(l0 + 3, 1)
            return carry

        lax.fori_loop(0, (GRP - 4) // 2 + 1, body, 0)
        finish(GRP - 2, 0)
        finish(GRP - 1, 1)
    plsc.subcore_barrier()

    @pl.when(sid < ATILES)
    def _out():
        pltpu.sync_copy(agg_sh.at[pl.ds(sid * ASTRIPE, ASTRIPE)],
                        out_hbm.at[cid, pl.ds(sid * ASTRIPE, ASTRIPE)])


_agg_call = functools.partial(
    pl.kernel,
    out_type=jax.ShapeDtypeStruct((NC, N, D), _f32),
    mesh=_mesh,
    scratch_types=[
        pltpu.VMEM((GRP, CH), jnp.int32),
        pltpu.VMEM((GRP, CH), jnp.int32),
        pltpu.VMEM((2, CH, D), _f32),
        pltpu.VMEM((AZ, D), _f32),
        pltpu.VMEM_SHARED((N, D), _f32),
        pltpu.SemaphoreType.DMA,
        pltpu.SemaphoreType.DMA,
    ],
)(_agg_body)


BN = 1000


def _prep_body(x_ref, dop_ref, o_ref):
    do = jnp.maximum(dop_ref[0] + dop_ref[1], 1.0)
    o_ref[...] = x_ref[...] * lax.rsqrt(do)


_prep_call = pl.pallas_call(
    _prep_body,
    grid=(N // BN,),
    in_specs=[
        pl.BlockSpec((BN, D), lambda i: (i, 0)),
        pl.BlockSpec((NC, BN, 1), lambda i: (0, i, 0)),
    ],
    out_specs=pl.BlockSpec((BN, D), lambda i: (i, 0)),
    out_shape=jax.ShapeDtypeStruct((N, D), _f32),
)


def _dense_math(aggp, dip, x, w, b, g, be):
    rs_in = lax.rsqrt(jnp.maximum(dip[0] + dip[1], 1.0))
    a = (aggp[0] + aggp[1]) * rs_in
    h = jnp.dot(a, w[...], preferred_element_type=_f32) + b[...]
    mu = jnp.mean(h, axis=1, keepdims=True)
    var = jnp.mean((h - mu) ** 2, axis=1, keepdims=True)
    h = (h - mu) * lax.rsqrt(var + 1e-5) * g[...] + be[...]
    return jnp.maximum(h, 0.0) + x[...]


def _dense_body2(aggp, dip, dop, x, w, b, g, be, out_h, out_s):
    h = _dense_math(aggp, dip, x, w, b, g, be)
    out_h[...] = h
    out_s[...] = h * lax.rsqrt(jnp.maximum(dop[0] + dop[1], 1.0))


def _dense_body1(aggp, dip, dop, x, w, b, g, be, out_h):
    out_h[...] = _dense_math(aggp, dip, x, w, b, g, be)


def _make_dense(n_out):
    body = _dense_body2 if n_out == 2 else _dense_body1
    out_shape = [jax.ShapeDtypeStruct((N, D), _f32)] * n_out
    out_specs = [pl.BlockSpec((BN, D), lambda i: (i, 0))] * n_out
    if n_out == 1:
        out_shape, out_specs = out_shape[0], out_specs[0]
    return pl.pallas_call(
        body,
        grid=(N // BN,),
        in_specs=[
            pl.BlockSpec((NC, BN, D), lambda i: (0, i, 0)),
            pl.BlockSpec((NC, BN, 1), lambda i: (0, i, 0)),
            pl.BlockSpec((NC, BN, 1), lambda i: (0, i, 0)),
            pl.BlockSpec((BN, D), lambda i: (i, 0)),
            pl.BlockSpec((D, D), lambda i: (0, 0)),
            pl.BlockSpec((1, D), lambda i: (0, 0)),
            pl.BlockSpec((1, D), lambda i: (0, 0)),
            pl.BlockSpec((1, D), lambda i: (0, 0)),
        ],
        out_specs=out_specs,
        out_shape=out_shape,
    )


_dense2 = _make_dense(2)
_dense1 = _make_dense(1)


def kernel(features, edge_index, W1, b1, ln_g1, ln_b1, W2, b2, ln_g2, ln_b2):
    src4d = edge_index[0].reshape(NW, NGRP, GRP, CH)
    dst4d = edge_index[1].reshape(NW, NGRP, GRP, CH)
    ones1 = jnp.ones((CH,), _f32)
    z1 = jnp.zeros((DEG_STRIPE,), _f32)
    z2 = jnp.zeros((AZ, D), _f32)

    src3d = edge_index[0].reshape(NW, NCH_W, CH)
    dst3d = edge_index[1].reshape(NW, NCH_W, CH)
    dout_p, din_p = _deg_call(src3d, dst3d, ones1, z1)
    dop = dout_p.reshape(NC, N, 1)
    dip = din_p.reshape(NC, N, 1)

    scaled = _prep_call(features, dop)
    aggp = _agg_call(src4d, dst4d, scaled, z2)
    h1, scaled2 = _dense2(aggp, dip, dop, features,
                          W1, b1.reshape(1, D), ln_g1.reshape(1, D),
                          ln_b1.reshape(1, D))
    aggp2 = _agg_call(src4d, dst4d, scaled2, z2)
    return _dense1(aggp2, dip, dop, h1,
                   W2, b2.reshape(1, D), ln_g2.reshape(1, D),
                   ln_b2.reshape(1, D))
